# Initial kernel scaffold; baseline (speedup 1.0000x reference)
import jax
import jax.numpy as jnp
from jax.experimental import pallas as pl


def kernel(x, batch_idx, condition, params):
    raise NotImplementedError("write your pallas kernel here")


# speedup vs baseline: 455.7818x; 455.7818x over previous
import jax
import jax.numpy as jnp
from jax.experimental import pallas as pl
from jax.experimental.pallas import tpu as pltpu

_NODES = [30, 6, 2, 1]
_FEATS = [3, 6, 12, 18]
_LAT = 10
_DISC = 5
_KNN = 15
_G = 128

_LVL_NAMES = [
    'se_W', 'se_b',
    'g0_W1', 'g0_b1', 'g0_W2', 'g0_b2',
    'g1_W1', 'g1_b1', 'g1_W2', 'g1_b2',
    'oe_W', 'oe_b', 'de_W', 'de_b', 'd_W', 'd_b', 'pool_w',
]


def _linear(xs, W_ref, b_ref, fin, fout, relu):
    out = []
    for o in range(fout):
        acc = xs[0] * W_ref[0, o]
        for f in range(1, fin):
            acc = acc + xs[f] * W_ref[f, o]
        acc = acc + b_ref[o]
        if relu:
            acc = jnp.maximum(acc, 0.0)
        out.append(acc)
    return out


def _body(x_ref, *refs):
    out_ref = refs[-1]
    wrefs = refs[:-1]
    lvls = []
    i = 0
    for _ in range(3):
        d = {}
        for nm in _LVL_NAMES:
            d[nm] = wrefs[i]
            i += 1
        lvls.append(d)
    last_W, last_b = wrefs[i], wrefs[i + 1]

    f32 = jnp.float32
    G = out_ref.shape[1]
    disc = jnp.zeros((1, G), f32)

    n = _NODES[0]
    xs = [x_ref[c] for c in range(_FEATS[0])]

    for l in range(3):
        p = lvls[l]
        fin = _FEATS[l]
        fnext = _FEATS[l + 1]
        ratio = _NODES[l + 1]

        xl = _linear(xs, p['se_W'], p['se_b'], fin, _LAT, relu=True)

        if l == 0:
            px, py = xl[0], xl[1]
            iota = jax.lax.broadcasted_iota(jnp.int32, (n, G), 0)
            D = []
            for k in range(n):
                dx = px - px[k:k + 1, :]
                dy = py - py[k:k + 1, :]
                d = dx * dx + dy * dy
                D.append(jnp.where(iota == k, 1e9, d))
            cnt = [jnp.full((n, G), float(n - 1 - j), f32) for j in range(n)]
            for j in range(n):
                Dj = D[j]
                for k in range(j + 1, n):
                    a = jnp.where(Dj <= D[k], 1.0, 0.0)
                    cnt[k] = cnt[k] + a
                    cnt[j] = cnt[j] - a
            M = [jnp.where(c < float(_KNN), 1.0, 0.0) for c in cnt]

        for layer in range(2):
            W1 = p['g%d_W1' % layer]; b1 = p['g%d_b1' % layer]
            W2 = p['g%d_W2' % layer]; b2 = p['g%d_b2' % layer]
            if l == 0:
                aggr = []
                for f in range(_LAT):
                    acc = M[0] * xl[f][0:1, :]
                    for j in range(1, n):
                        acc = acc + M[j] * xl[f][j:j + 1, :]
                    aggr.append(acc)
            else:
                aggr = [jnp.sum(xl[f], axis=0, keepdims=True) - xl[f]
                        for f in range(_LAT)]
            h = [xl[f] + aggr[f] for f in range(_LAT)]
            t = _linear(h, W1, b1, _LAT, 2 * _LAT, relu=True)
            u = _linear(t, W2, b2, 2 * _LAT, _LAT, relu=False)
            xl = [u[f] + xl[f] for f in range(_LAT)]

        x_emb = _linear(xl, p['oe_W'], p['oe_b'], _LAT, fnext, relu=True)
        a = _linear(xl, p['de_W'], p['de_b'], _LAT, _DISC, relu=True)
        a = [a[dd] + xl[dd] for dd in range(_DISC)]
        for dd in range(_DISC):
            pooled = jnp.sum(a[dd], axis=0, keepdims=True)
            disc = disc + pooled * p['d_W'][dd, 0]
        disc = disc + p['d_b'][0]

        s = x_emb[0] * p['pool_w'][0]
        for e in range(1, fnext):
            s = s + x_emb[e] * p['pool_w'][e]
        th = jnp.tanh(s)
        y = [x_emb[e] * th for e in range(fnext)]

        iota_n = jax.lax.broadcasted_iota(jnp.int32, (n, G), 0)
        cols = []
        for j in range(n):
            sj = s[j:j + 1, :]
            cond = (s > sj) | ((s == sj) & (iota_n < j))
            contrib = jnp.where(cond, 1.0, 0.0)
            cols.append(jnp.sum(contrib, axis=0, keepdims=True))
        R = jnp.concatenate(cols, axis=0)

        nxs = []
        for e in range(fnext):
            rows = []
            for r0 in range(ratio):
                sel = jnp.where(R == float(r0), 1.0, 0.0)
                rows.append(jnp.sum(sel * y[e], axis=0, keepdims=True))
            nxs.append(jnp.concatenate(rows, axis=0) if ratio > 1 else rows[0])
        xs = nxs
        n = ratio

    for e in range(_FEATS[3]):
        disc = disc + xs[e] * last_W[e, 0]
    disc = disc + last_b[0]
    out_ref[...] = disc


def kernel(x, batch_idx, condition, params):
    del batch_idx, condition
    B = x.shape[0] // _NODES[0]
    G = _G if B % _G == 0 else B
    xt = x.reshape(B, _NODES[0], _FEATS[0]).transpose(2, 1, 0)

    flat = []
    for l in range(3):
        p = params['lvl%d' % l]
        for nm in _LVL_NAMES:
            w = p[nm]
            if nm == 'pool_w':
                w = w / jnp.linalg.norm(w)
            flat.append(w)
    flat.append(params['last_W'])
    flat.append(params['last_b'])

    smem = pl.BlockSpec(memory_space=pltpu.SMEM)
    out = pl.pallas_call(
        _body,
        grid=(B // G,),
        in_specs=[pl.BlockSpec((_FEATS[0], _NODES[0], G), lambda i: (0, 0, i))]
        + [smem] * len(flat),
        out_specs=pl.BlockSpec((1, G), lambda i: (0, i)),
        out_shape=jax.ShapeDtypeStruct((1, B), jnp.float32),
        compiler_params=pltpu.CompilerParams(
            dimension_semantics=("parallel",)),
    )(xt, *flat)
    return out.reshape(B, 1)
